# baseline (speedup 1.0000x reference)
import functools

import jax
import jax.numpy as jnp
from jax import lax
from jax.experimental import pallas as pl
from jax.experimental.pallas import tpu as pltpu
from jax.experimental.pallas import tpu_sc as plsc

_D = 32
_L = 128
_CB = 4
_U = _CB * _L
_NC = 2
_NS = 16
_NW = _NC * _NS
_RUN = 8 * _U


@functools.lru_cache(maxsize=None)
def _build(batch: int, hist: int):
    n_bt = batch // _L
    n_cb = n_bt // _CB
    n_units = hist * n_cb
    units_per_w = n_units // _NW
    n_rt = _D // 8
    out_words = hist * _D * batch
    assert n_bt * _L == batch and n_cb * _CB == n_bt
    assert units_per_w * _NW == n_units and units_per_w % 2 == 0

    mesh = plsc.VectorSubcoreMesh(core_axis_name="c", subcore_axis_name="s")

    @functools.partial(
        pl.kernel,
        mesh=mesh,
        compiler_params=pltpu.CompilerParams(
            use_tc_tiling_on_sc=False, needs_layout_passes=False),
        out_type=jax.ShapeDtypeStruct((out_words,), jnp.float32),
        scratch_types=[
            pltpu.VMEM((2, _U), jnp.int32),
            pltpu.VMEM((2, _U, _D), jnp.float32),
            pltpu.VMEM((2, _D * _U), jnp.float32),
            pltpu.SemaphoreType.DMA,
            pltpu.SemaphoreType.DMA,
            pltpu.SemaphoreType.DMA,
            pltpu.SemaphoreType.DMA,
            pltpu.SemaphoreType.DMA,
            pltpu.SemaphoreType.DMA,
        ],
    )
    def gather_kernel(table_hbm, xt_hbm, out_hbm, idxb, slots, stage,
                      gi0, gi1, g0, g1, s0, s1):
        wid = lax.axis_index("s") * _NC + lax.axis_index("c")
        isems = (gi0, gi1)
        gsems = (g0, g1)
        ssems = (s0, s1)
        iota = lax.iota(jnp.int32, 16)
        dvec = [iota + 16 * k for k in range(_D // 16)]
        sbase = [
            lax.shift_right_logical(d, 3) * _RUN + lax.bitwise_and(d, 7) * _L
            for d in dvec
        ]

        def fire_idx(i, p):
            h = lax.div(i, n_cb)
            cb = lax.rem(i, n_cb)
            pltpu.async_copy(
                xt_hbm.at[h, pl.ds(cb * _U, _U)], idxb.at[p], isems[p])

        def fire_gathers(p):
            pltpu.make_async_copy(
                xt_hbm.at[0, pl.ds(0, _U)], idxb.at[p], isems[p]).wait()
            for b in range(_CB):
                pltpu.async_copy(
                    table_hbm.at[idxb.at[p, pl.ds(b * _L, _L)]],
                    slots.at[p, pl.ds(b * _L, _L)], gsems[p])

        def drain_gathers(p):
            pltpu.make_async_copy(
                table_hbm.at[pl.ds(0, _U)], slots.at[p], gsems[p]).wait()

        def transpose(p):
            for q in range(_CB):
                qoff = q * 8 * _L

                def tbody(li, idxs):
                    new = list(idxs)
                    for w in range(4):
                        l = q * _L + 4 * li + w
                        for k in range(_D // 16):
                            v = slots[p, l, pl.ds(16 * k, 16)]
                            plsc.store_scatter(stage.at[p], [new[k]], v)
                            new[k] = new[k] + 1
                    return tuple(new)

                lax.fori_loop(0, _L // 4, tbody,
                              tuple(b + qoff for b in sbase))

        def fire_stores(i, p):
            h = lax.div(i, n_cb)
            cb = lax.rem(i, n_cb)
            for r in range(n_rt):
                off = (h * n_rt + r) * n_bt * _L * 8 + cb * _RUN
                pltpu.async_copy(
                    stage.at[p, pl.ds(r * _RUN, _RUN)],
                    out_hbm.at[pl.ds(off, _RUN)], ssems[p])

        def wait_stores(p):
            for r in range(n_rt):
                pltpu.make_async_copy(
                    stage.at[p, pl.ds(0, _RUN)],
                    out_hbm.at[pl.ds(0, _RUN)], ssems[p]).wait()

        u0 = wid * units_per_w
        fire_idx(u0, 0)
        fire_gathers(0)
        fire_idx(u0 + 1, 1)

        def pair_body(t, carry):
            i0 = u0 + 2 * t

            fire_gathers(1)
            drain_gathers(0)

            @pl.when(t > 0)
            def _():
                wait_stores(0)

            transpose(0)
            fire_stores(i0, 0)

            @pl.when(t + 1 < units_per_w // 2)
            def _():
                fire_idx(i0 + 2, 0)

            @pl.when(t + 1 < units_per_w // 2)
            def _():
                fire_gathers(0)
            drain_gathers(1)

            @pl.when(t > 0)
            def _():
                wait_stores(1)

            transpose(1)
            fire_stores(i0 + 1, 1)

            @pl.when(t + 1 < units_per_w // 2)
            def _():
                fire_idx(i0 + 3, 1)
            return carry

        lax.fori_loop(0, units_per_w // 2, pair_body, 0)
        wait_stores(0)
        wait_stores(1)

    return gather_kernel


def kernel(x, table):
    b, h = x.shape
    v, d = table.shape
    table128 = table.reshape(v // 4, 4 * d)
    idx = x.T.astype(jnp.int32)
    flat = _build(b, h)(table128.reshape(v, d), idx)
    out5 = flat.reshape(h, _D // 8, b // _L, 8, _L)
    return out5.transpose((2, 4, 0, 1, 3)).reshape(b, h, _D)


# speedup vs baseline: 1.0130x; 1.0130x over previous
import functools

import jax
import jax.numpy as jnp
from jax import lax
from jax.experimental import pallas as pl
from jax.experimental.pallas import tpu as pltpu
from jax.experimental.pallas import tpu_sc as plsc

_D = 32
_L = 128
_CB = 4
_U = _CB * _L
_NC = 2
_NS = 16
_NW = _NC * _NS
_RUN = 8 * _U


@functools.lru_cache(maxsize=None)
def _build(batch: int, hist: int):
    n_bt = batch // _L
    n_cb = n_bt // _CB
    n_units = hist * n_cb
    units_per_w = n_units // _NW
    n_rt = _D // 8
    out_words = hist * _D * batch
    assert n_bt * _L == batch and n_cb * _CB == n_bt
    assert units_per_w * _NW == n_units and units_per_w % 2 == 0

    mesh = plsc.VectorSubcoreMesh(core_axis_name="c", subcore_axis_name="s")

    @functools.partial(
        pl.kernel,
        mesh=mesh,
        compiler_params=pltpu.CompilerParams(
            use_tc_tiling_on_sc=False, needs_layout_passes=False),
        out_type=jax.ShapeDtypeStruct((out_words,), jnp.float32),
        scratch_types=[
            pltpu.VMEM((2, _U), jnp.int32),
            pltpu.VMEM((2, _U, _D), jnp.float32),
            pltpu.VMEM((2, _D * _U), jnp.float32),
            pltpu.SemaphoreType.DMA,
            pltpu.SemaphoreType.DMA,
            pltpu.SemaphoreType.DMA,
            pltpu.SemaphoreType.DMA,
            pltpu.SemaphoreType.DMA,
            pltpu.SemaphoreType.DMA,
        ],
    )
    def gather_kernel(table_hbm, xt_hbm, out_hbm, idxb, slots, stage,
                      gi0, gi1, g0, g1, s0, s1):
        wid = lax.axis_index("s") * _NC + lax.axis_index("c")
        isems = (gi0, gi1)
        gsems = (g0, g1)
        ssems = (s0, s1)
        iota = lax.iota(jnp.int32, 16)
        dvec = [iota + 16 * k for k in range(_D // 16)]
        sbase = [
            lax.shift_right_logical(d, 3) * _RUN + lax.bitwise_and(d, 7) * _L
            for d in dvec
        ]

        def fire_idx(i, p):
            h = lax.div(i, n_cb)
            cb = lax.rem(i, n_cb)
            pltpu.async_copy(
                xt_hbm.at[h, pl.ds(cb * _U, _U)], idxb.at[p], isems[p])

        def fire_gathers(p):
            pltpu.make_async_copy(
                xt_hbm.at[0, pl.ds(0, _U)], idxb.at[p], isems[p]).wait()
            for b in range(_CB):
                pltpu.async_copy(
                    table_hbm.at[idxb.at[p, pl.ds(b * _L, _L)]],
                    slots.at[p, pl.ds(b * _L, _L)], gsems[p])

        def drain_gathers(p):
            pltpu.make_async_copy(
                table_hbm.at[pl.ds(0, _U)], slots.at[p], gsems[p]).wait()

        def transpose(p):
            for q in range(_CB):
                qoff = q * 8 * _L

                def tbody(li, idxs):
                    new = list(idxs)
                    for w in range(4):
                        l = q * _L + 4 * li + w
                        for k in range(_D // 16):
                            v = slots[p, l, pl.ds(16 * k, 16)]
                            plsc.store_scatter(stage.at[p], [new[k]], v)
                            new[k] = new[k] + 1
                    return tuple(new)

                lax.fori_loop(0, _L // 4, tbody,
                              tuple(b + qoff for b in sbase))

        def fire_stores(i, p):
            h = lax.div(i, n_cb)
            cb = lax.rem(i, n_cb)
            for r in range(n_rt):
                off = (h * n_rt + r) * n_bt * _L * 8 + cb * _RUN
                pltpu.async_copy(
                    stage.at[p, pl.ds(r * _RUN, _RUN)],
                    out_hbm.at[pl.ds(off, _RUN)], ssems[p])

        def wait_stores(p):
            for r in range(n_rt):
                pltpu.make_async_copy(
                    stage.at[p, pl.ds(0, _RUN)],
                    out_hbm.at[pl.ds(0, _RUN)], ssems[p]).wait()

        u0 = wid * units_per_w
        fire_idx(u0, 0)
        fire_gathers(0)
        fire_idx(u0 + 1, 1)

        def pair_body(t, carry):
            i0 = u0 + 2 * t

            fire_gathers(1)
            drain_gathers(0)

            @pl.when(t > 0)
            def _():
                wait_stores(0)

            transpose(0)
            fire_stores(i0, 0)

            @pl.when(t + 1 < units_per_w // 2)
            def _():
                fire_idx(i0 + 2, 0)

            @pl.when(t + 1 < units_per_w // 2)
            def _():
                fire_gathers(0)
            drain_gathers(1)

            @pl.when(t > 0)
            def _():
                wait_stores(1)

            transpose(1)
            fire_stores(i0 + 1, 1)

            @pl.when(t + 1 < units_per_w // 2)
            def _():
                fire_idx(i0 + 3, 1)
            return carry

        lax.fori_loop(0, units_per_w // 2, pair_body, 0)
        wait_stores(0)
        wait_stores(1)

    return gather_kernel


def kernel(x, table):
    b, h = x.shape
    v, d = table.shape
    table4 = jnp.pad(table, ((0, 0), (0, 128 - d))).reshape(4 * v, d)
    idx4 = x.T.astype(jnp.int32) * 4
    flat = _build(b, h)(table4, idx4)
    out5 = flat.reshape(h, _D // 8, b // _L, 8, _L)
    return out5.transpose((2, 4, 0, 1, 3)).reshape(b, h, _D)
